# baseline (speedup 1.0000x reference)
import functools

import jax
import jax.numpy as jnp
from jax import lax
from jax.experimental import pallas as pl
from jax.experimental.pallas import tpu as pltpu
from jax.experimental.pallas import tpu_sc as plsc

N_NODES = 10000
N_EDGES = 320000
D = 128

NC = 2
NS = 16
NW = NC * NS
LANE = 128
CH = 80
E_PAD = NW * CH * LANE
N_PAD = 10240
RPT = N_PAD // NS
DEG_W = 16

_mesh = plsc.VectorSubcoreMesh(
    core_axis_name="c", subcore_axis_name="s", num_cores=NC, num_subcores=NS)


@functools.partial(
    pl.kernel,
    out_type=jax.ShapeDtypeStruct((NC, N_PAD, DEG_W), jnp.float32),
    mesh=_mesh,
    scratch_types=[
        pltpu.VMEM((CH, LANE), jnp.int32),
        pltpu.VMEM((LANE, DEG_W), jnp.float32),
        pltpu.VMEM((LANE, DEG_W), jnp.float32),
        pltpu.VMEM_SHARED((N_PAD, DEG_W), jnp.float32),
    ],
)
def _deg_kernel(dstp_hbm, out_hbm, didx, ones, zeros, acc):
    c = lax.axis_index("c")
    s = lax.axis_index("s")
    wid = c * NS + s

    one16 = jnp.ones((16,), jnp.float32)
    zero16 = jnp.zeros((16,), jnp.float32)

    def fill(i, _):
        ones[i, pl.ds(0, 16)] = one16
        zeros[i, pl.ds(0, 16)] = zero16
        return 0
    lax.fori_loop(0, LANE, fill, 0)

    base = s * RPT
    for k in range(RPT // LANE):
        pltpu.sync_copy(zeros, acc.at[pl.ds(base + k * LANE, LANE)])
    plsc.subcore_barrier()

    pltpu.sync_copy(dstp_hbm.at[wid], didx)

    def body(j, _):
        pltpu.sync_copy(ones, acc.at[didx.at[j]], add=True)
        return 0
    lax.fori_loop(0, CH, body, 0)

    plsc.subcore_barrier()
    pltpu.sync_copy(acc.at[pl.ds(base, RPT)],
                    out_hbm.at[c, pl.ds(base, RPT)])


@functools.partial(
    pl.kernel,
    out_type=jax.ShapeDtypeStruct((NC, N_PAD, D), jnp.float32),
    mesh=_mesh,
    scratch_types=[
        pltpu.VMEM((CH, LANE), jnp.int32),
        pltpu.VMEM((CH, LANE), jnp.int32),
        pltpu.VMEM((LANE, D), jnp.float32),
        pltpu.VMEM_SHARED((N_PAD, D), jnp.float32),
        pltpu.SemaphoreType.DMA,
    ],
)
def _hop_kernel(h_hbm, srcp_hbm, dstp_hbm, out_hbm, sidx, didx,
                rows, acc, sem0):
    c = lax.axis_index("c")
    s = lax.axis_index("s")
    wid = c * NS + s

    zero16 = jnp.zeros((16,), jnp.float32)

    with jax.named_scope("hop_zerofill"):
        def fill(i, _):
            def fill_in(j, _):
                rows[i, pl.ds(j * 16, 16)] = zero16
                return 0
            lax.fori_loop(0, D // 16, fill_in, 0)
            return 0
        lax.fori_loop(0, LANE, fill, 0)

    base = s * RPT
    with jax.named_scope("hop_zeroacc"):
        for k in range(RPT // LANE):
            pltpu.sync_copy(rows, acc.at[pl.ds(base + k * LANE, LANE)])
        plsc.subcore_barrier()

    with jax.named_scope("hop_idxload"):
        pltpu.sync_copy(srcp_hbm.at[wid], sidx)
        pltpu.sync_copy(dstp_hbm.at[wid], didx)

    with jax.named_scope("hop_edges"):
        def body(j, _):
            pltpu.async_copy(h_hbm.at[sidx.at[j]], rows, sem0).wait()
            pltpu.sync_copy(rows, acc.at[didx.at[j]], add=True)
            return 0
        lax.fori_loop(0, CH, body, 0)
        plsc.subcore_barrier()

    with jax.named_scope("hop_writeout"):
        pltpu.sync_copy(acc.at[pl.ds(base, RPT)],
                        out_hbm.at[c, pl.ds(base, RPT)])


_BLK = 1024


def _scale_body(x_ref, d_ref, y_ref):
    deg = d_ref[0, :, 0:1] + d_ref[1, :, 0:1] + 1.0
    y_ref[...] = x_ref[...] * lax.rsqrt(deg)


def _mid_body(zp_ref, y_ref, d_ref, z_ref):
    deg = d_ref[0, :, 0:1] + d_ref[1, :, 0:1] + 1.0
    z_ref[...] = (zp_ref[0] + zp_ref[1] + y_ref[...]) / deg


def _out_body(wp_ref, z_ref, d_ref, w_ref, b_ref, o_ref):
    deg = d_ref[0, :, 0:1] + d_ref[1, :, 0:1] + 1.0
    w = (wp_ref[0] + wp_ref[1] + z_ref[...]) * lax.rsqrt(deg)
    o_ref[...] = (jnp.dot(w, w_ref[...], preferred_element_type=jnp.float32)
                  + b_ref[...])


def _row_spec(blk=_BLK, width=D):
    return pl.BlockSpec((blk, width), lambda i: (i, 0))


def _pair_spec(blk=_BLK, width=D):
    return pl.BlockSpec((2, blk, width), lambda i: (0, i, 0))


_GRID = (N_PAD // _BLK,)

_scale = pl.pallas_call(
    _scale_body,
    grid=_GRID,
    in_specs=[_row_spec(), _pair_spec(width=DEG_W)],
    out_specs=_row_spec(),
    out_shape=jax.ShapeDtypeStruct((N_PAD, D), jnp.float32),
)

_mid = pl.pallas_call(
    _mid_body,
    grid=_GRID,
    in_specs=[_pair_spec(), _row_spec(), _pair_spec(width=DEG_W)],
    out_specs=_row_spec(),
    out_shape=jax.ShapeDtypeStruct((N_PAD, D), jnp.float32),
)

_out = pl.pallas_call(
    _out_body,
    grid=_GRID,
    in_specs=[
        _pair_spec(), _row_spec(), _pair_spec(width=DEG_W),
        pl.BlockSpec((D, D), lambda i: (0, 0)),
        pl.BlockSpec((1, D), lambda i: (0, 0)),
    ],
    out_specs=_row_spec(),
    out_shape=jax.ShapeDtypeStruct((N_PAD, D), jnp.float32),
)


def kernel(x, edge_index, W, b):
    src = edge_index[0].astype(jnp.int32)
    dst = edge_index[1].astype(jnp.int32)
    pad = jnp.full((E_PAD - N_EDGES,), N_NODES, jnp.int32)
    srcp = jnp.concatenate([src, pad]).reshape(NW, CH, LANE)
    dstp = jnp.concatenate([dst, pad]).reshape(NW, CH, LANE)
    x_p = jnp.pad(x, ((0, N_PAD - N_NODES), (0, 0)))
    degp = _deg_kernel(dstp)
    y = _scale(x_p, degp)
    zp = _hop_kernel(y, srcp, dstp)
    z = _mid(zp, y, degp)
    wp = _hop_kernel(z, srcp, dstp)
    out = _out(wp, z, degp, W, b.reshape(1, D))
    return out[:N_NODES]


# speedup vs baseline: 2.8511x; 2.8511x over previous
import functools

import jax
import jax.numpy as jnp
from jax import lax
from jax.experimental import pallas as pl
from jax.experimental.pallas import tpu as pltpu
from jax.experimental.pallas import tpu_sc as plsc

N_NODES = 10000
N_EDGES = 320000
D = 128

NC = 2
NS = 16
NW = NC * NS
LANE = 128
CH = 80
E_PAD = NW * CH * LANE
N_PAD = 10240
RPT = N_PAD // NS
DEG_W = 16

_mesh = plsc.VectorSubcoreMesh(
    core_axis_name="c", subcore_axis_name="s", num_cores=NC, num_subcores=NS)


@functools.partial(
    pl.kernel,
    out_type=jax.ShapeDtypeStruct((NC, N_PAD, DEG_W), jnp.float32),
    mesh=_mesh,
    scratch_types=[
        pltpu.VMEM((CH, LANE), jnp.int32),
        pltpu.VMEM((LANE, DEG_W), jnp.float32),
        pltpu.VMEM((LANE, DEG_W), jnp.float32),
        pltpu.VMEM_SHARED((N_PAD, DEG_W), jnp.float32),
    ],
)
def _deg_kernel(dstp_hbm, out_hbm, didx, ones, zeros, acc):
    c = lax.axis_index("c")
    s = lax.axis_index("s")
    wid = c * NS + s

    one16 = jnp.ones((16,), jnp.float32)
    zero16 = jnp.zeros((16,), jnp.float32)

    def fill(i, _):
        ones[i, pl.ds(0, 16)] = one16
        zeros[i, pl.ds(0, 16)] = zero16
        return 0
    lax.fori_loop(0, LANE, fill, 0)

    base = s * RPT
    for k in range(RPT // LANE):
        pltpu.sync_copy(zeros, acc.at[pl.ds(base + k * LANE, LANE)])
    plsc.subcore_barrier()

    pltpu.sync_copy(dstp_hbm.at[wid], didx)

    def body(j, _):
        pltpu.sync_copy(ones, acc.at[didx.at[j]], add=True)
        return 0
    lax.fori_loop(0, CH, body, 0)

    plsc.subcore_barrier()
    pltpu.sync_copy(acc.at[pl.ds(base, RPT)],
                    out_hbm.at[c, pl.ds(base, RPT)])


@functools.partial(
    pl.kernel,
    out_type=jax.ShapeDtypeStruct((NC, N_PAD, D), jnp.float32),
    mesh=_mesh,
    scratch_types=[
        pltpu.VMEM((CH, LANE), jnp.int32),
        pltpu.VMEM((CH, LANE), jnp.int32),
        pltpu.VMEM((LANE, D), jnp.float32),
        pltpu.VMEM_SHARED((N_PAD, D), jnp.float32),
        pltpu.SemaphoreType.DMA,
    ],
)
def _hop_kernel(h_hbm, srcp_hbm, dstp_hbm, out_hbm, sidx, didx,
                rows, acc, sem0):
    c = lax.axis_index("c")
    s = lax.axis_index("s")
    wid = c * NS + s

    zero16 = jnp.zeros((16,), jnp.float32)

    with jax.named_scope("hop_zerofill"):
        def fill(i, _):
            def fill_in(j, _):
                rows[i, pl.ds(j * 16, 16)] = zero16
                return 0
            lax.fori_loop(0, D // 16, fill_in, 0)
            return 0
        lax.fori_loop(0, LANE, fill, 0)

    base = s * RPT
    with jax.named_scope("hop_zeroacc"):
        for k in range(RPT // LANE):
            pltpu.sync_copy(rows, acc.at[pl.ds(base + k * LANE, LANE)])
        plsc.subcore_barrier()

    with jax.named_scope("hop_idxload"):
        pltpu.sync_copy(srcp_hbm.at[wid], sidx)
        pltpu.sync_copy(dstp_hbm.at[wid], didx)

    with jax.named_scope("hop_edges"):
        def body(j, _):
            pltpu.async_copy(h_hbm.at[sidx.at[j]], rows, sem0).wait()
            pltpu.sync_copy(rows, acc.at[didx.at[j]], add=True)
            return 0
        lax.fori_loop(0, CH, body, 0)
        plsc.subcore_barrier()

    with jax.named_scope("hop_writeout"):
        pltpu.sync_copy(acc.at[pl.ds(base, RPT)],
                        out_hbm.at[c, pl.ds(base, RPT)])


_BLK = 1024


def _scale_body(x_ref, d_ref, y_ref):
    deg = d_ref[0, :, 0:1] + d_ref[1, :, 0:1] + 1.0
    y_ref[...] = x_ref[...] * lax.rsqrt(deg)


def _mid_body(zp_ref, y_ref, d_ref, z_ref):
    deg = d_ref[0, :, 0:1] + d_ref[1, :, 0:1] + 1.0
    z_ref[...] = (zp_ref[0] + zp_ref[1] + y_ref[...]) / deg


def _out_body(wp_ref, z_ref, d_ref, w_ref, b_ref, o_ref):
    deg = d_ref[0, :, 0:1] + d_ref[1, :, 0:1] + 1.0
    w = (wp_ref[0] + wp_ref[1] + z_ref[...]) * lax.rsqrt(deg)
    o_ref[...] = (jnp.dot(w, w_ref[...], preferred_element_type=jnp.float32)
                  + b_ref[...])


def _row_spec(blk=_BLK, width=D):
    return pl.BlockSpec((blk, width), lambda i: (i, 0))


def _pair_spec(blk=_BLK, width=D):
    return pl.BlockSpec((2, blk, width), lambda i: (0, i, 0))


_GRID = (N_PAD // _BLK,)

_scale = pl.pallas_call(
    _scale_body,
    grid=_GRID,
    in_specs=[_row_spec(), _pair_spec(width=DEG_W)],
    out_specs=_row_spec(),
    out_shape=jax.ShapeDtypeStruct((N_PAD, D), jnp.float32),
)

_mid = pl.pallas_call(
    _mid_body,
    grid=_GRID,
    in_specs=[_pair_spec(), _row_spec(), _pair_spec(width=DEG_W)],
    out_specs=_row_spec(),
    out_shape=jax.ShapeDtypeStruct((N_PAD, D), jnp.float32),
)

_out = pl.pallas_call(
    _out_body,
    grid=_GRID,
    in_specs=[
        _pair_spec(), _row_spec(), _pair_spec(width=DEG_W),
        pl.BlockSpec((D, D), lambda i: (0, 0)),
        pl.BlockSpec((1, D), lambda i: (0, 0)),
    ],
    out_specs=_row_spec(),
    out_shape=jax.ShapeDtypeStruct((N_PAD, D), jnp.float32),
)


def kernel(x, edge_index, W, b):
    src = edge_index[0].astype(jnp.int32)
    dst = edge_index[1].astype(jnp.int32)
    pad = N_NODES + (jnp.arange(E_PAD - N_EDGES, dtype=jnp.int32)
                     % (N_PAD - N_NODES))
    srcp = jnp.concatenate([src, pad]).reshape(NW, CH, LANE)
    dstp = jnp.concatenate([dst, pad]).reshape(NW, CH, LANE)
    x_p = jnp.pad(x, ((0, N_PAD - N_NODES), (0, 0)))
    degp = _deg_kernel(dstp)
    y = _scale(x_p, degp)
    zp = _hop_kernel(y, srcp, dstp)
    z = _mid(zp, y, degp)
    wp = _hop_kernel(z, srcp, dstp)
    out = _out(wp, z, degp, W, b.reshape(1, D))
    return out[:N_NODES]


# speedup vs baseline: 3.6218x; 1.2703x over previous
import functools

import jax
import jax.numpy as jnp
from jax import lax
from jax.experimental import pallas as pl
from jax.experimental.pallas import tpu as pltpu
from jax.experimental.pallas import tpu_sc as plsc

N_NODES = 10000
N_EDGES = 320000
D = 128

NC = 2
NS = 16
NW = NC * NS
LANE = 128
CH = 80
E_PAD = NW * CH * LANE
N_PAD = 10240
RPT = N_PAD // NS
DEG_W = 16

_mesh = plsc.VectorSubcoreMesh(
    core_axis_name="c", subcore_axis_name="s", num_cores=NC, num_subcores=NS)


@functools.partial(
    pl.kernel,
    out_type=jax.ShapeDtypeStruct((NC, N_PAD, DEG_W), jnp.float32),
    mesh=_mesh,
    scratch_types=[
        pltpu.VMEM((CH, LANE), jnp.int32),
        pltpu.VMEM((LANE, DEG_W), jnp.float32),
        pltpu.VMEM((LANE, DEG_W), jnp.float32),
        pltpu.VMEM_SHARED((N_PAD, DEG_W), jnp.float32),
    ],
)
def _deg_kernel(dstp_hbm, out_hbm, didx, ones, zeros, acc):
    c = lax.axis_index("c")
    s = lax.axis_index("s")
    wid = c * NS + s

    one16 = jnp.ones((16,), jnp.float32)
    zero16 = jnp.zeros((16,), jnp.float32)

    def fill(i, _):
        ones[i, pl.ds(0, 16)] = one16
        zeros[i, pl.ds(0, 16)] = zero16
        return 0
    lax.fori_loop(0, LANE, fill, 0)

    base = s * RPT
    for k in range(RPT // LANE):
        pltpu.sync_copy(zeros, acc.at[pl.ds(base + k * LANE, LANE)])
    plsc.subcore_barrier()

    pltpu.sync_copy(dstp_hbm.at[wid], didx)

    def body(j, _):
        pltpu.sync_copy(ones, acc.at[didx.at[j]], add=True)
        return 0
    lax.fori_loop(0, CH, body, 0)

    plsc.subcore_barrier()
    pltpu.sync_copy(acc.at[pl.ds(base, RPT)],
                    out_hbm.at[c, pl.ds(base, RPT)])


@functools.partial(
    pl.kernel,
    out_type=jax.ShapeDtypeStruct((NC, N_PAD, D), jnp.float32),
    mesh=_mesh,
    scratch_types=[
        pltpu.VMEM((2, LANE), jnp.int32),
        pltpu.VMEM((2, LANE), jnp.int32),
        pltpu.VMEM((LANE, D), jnp.float32),
        pltpu.VMEM((LANE, D), jnp.float32),
        pltpu.VMEM_SHARED((N_PAD, D), jnp.float32),
        pltpu.SemaphoreType.DMA,
        pltpu.SemaphoreType.DMA,
        pltpu.SemaphoreType.DMA,
        pltpu.SemaphoreType.DMA,
        pltpu.SemaphoreType.DMA,
        pltpu.SemaphoreType.DMA,
    ],
)
def _hop_kernel(h_hbm, idxc_hbm, out_hbm, slota, slotb, rowsa, rowsb,
                acc, sia, sib, sga, sgb, ssa, ssb):
    c = lax.axis_index("c")
    s = lax.axis_index("s")
    wid = c * NS + s
    cbase = wid * CH

    zero16 = jnp.zeros((16,), jnp.float32)

    with jax.named_scope("hop_zerofill"):
        def fill(i, _):
            def fill_in(j, _):
                rowsa[i, pl.ds(j * 16, 16)] = zero16
                return 0
            lax.fori_loop(0, D // 16, fill_in, 0)
            return 0
        lax.fori_loop(0, LANE, fill, 0)

    base = s * RPT
    with jax.named_scope("hop_zeroacc"):
        for k in range(RPT // LANE):
            pltpu.sync_copy(rowsa, acc.at[pl.ds(base + k * LANE, LANE)])
        plsc.subcore_barrier()

    with jax.named_scope("hop_edges"):
        def wait_idx(j, slot, sem):
            pltpu.make_async_copy(idxc_hbm.at[cbase + j], slot, sem).wait()

        def wait_gather(slot, buf, sem):
            pltpu.make_async_copy(h_hbm.at[slot.at[0]], buf, sem).wait()

        def wait_scatter(buf, slot, sem):
            pltpu.make_async_copy(buf, acc.at[slot.at[1]], sem).wait()

        pltpu.async_copy(idxc_hbm.at[cbase], slota, sia)
        wait_idx(0, slota, sia)
        pltpu.async_copy(h_hbm.at[slota.at[0]], rowsa, sga)

        def body(jj, _):
            j1 = 2 * jj + 1
            j2 = 2 * jj + 2

            @pl.when(jj > 0)
            def _():
                wait_scatter(rowsb, slotb, ssb)
            pltpu.async_copy(idxc_hbm.at[cbase + j1], slotb, sib)
            wait_gather(slota, rowsa, sga)
            pltpu.async_copy(rowsa, acc.at[slota.at[1]], ssa, add=True)
            wait_idx(j1, slotb, sib)
            pltpu.async_copy(h_hbm.at[slotb.at[0]], rowsb, sgb)
            wait_scatter(rowsa, slota, ssa)

            @pl.when(j2 < CH)
            def _():
                pltpu.async_copy(idxc_hbm.at[cbase + j2], slota, sia)
                wait_idx(j2, slota, sia)
            wait_gather(slotb, rowsb, sgb)
            pltpu.async_copy(rowsb, acc.at[slotb.at[1]], ssb, add=True)

            @pl.when(j2 < CH)
            def _():
                pltpu.async_copy(h_hbm.at[slota.at[0]], rowsa, sga)
            return 0
        lax.fori_loop(0, CH // 2, body, 0)
        wait_scatter(rowsb, slotb, ssb)
        plsc.subcore_barrier()

    with jax.named_scope("hop_writeout"):
        pltpu.sync_copy(acc.at[pl.ds(base, RPT)],
                        out_hbm.at[c, pl.ds(base, RPT)])


_BLK = 1024


def _scale_body(x_ref, d_ref, y_ref):
    deg = d_ref[0, :, 0:1] + d_ref[1, :, 0:1] + 1.0
    y_ref[...] = x_ref[...] * lax.rsqrt(deg)


def _mid_body(zp_ref, y_ref, d_ref, z_ref):
    deg = d_ref[0, :, 0:1] + d_ref[1, :, 0:1] + 1.0
    z_ref[...] = (zp_ref[0] + zp_ref[1] + y_ref[...]) / deg


def _out_body(wp_ref, z_ref, d_ref, w_ref, b_ref, o_ref):
    deg = d_ref[0, :, 0:1] + d_ref[1, :, 0:1] + 1.0
    w = (wp_ref[0] + wp_ref[1] + z_ref[...]) * lax.rsqrt(deg)
    o_ref[...] = (jnp.dot(w, w_ref[...], preferred_element_type=jnp.float32)
                  + b_ref[...])


def _row_spec(blk=_BLK, width=D):
    return pl.BlockSpec((blk, width), lambda i: (i, 0))


def _pair_spec(blk=_BLK, width=D):
    return pl.BlockSpec((2, blk, width), lambda i: (0, i, 0))


_GRID = (N_PAD // _BLK,)

_scale = pl.pallas_call(
    _scale_body,
    grid=_GRID,
    in_specs=[_row_spec(), _pair_spec(width=DEG_W)],
    out_specs=_row_spec(),
    out_shape=jax.ShapeDtypeStruct((N_PAD, D), jnp.float32),
)

_mid = pl.pallas_call(
    _mid_body,
    grid=_GRID,
    in_specs=[_pair_spec(), _row_spec(), _pair_spec(width=DEG_W)],
    out_specs=_row_spec(),
    out_shape=jax.ShapeDtypeStruct((N_PAD, D), jnp.float32),
)

_out = pl.pallas_call(
    _out_body,
    grid=_GRID,
    in_specs=[
        _pair_spec(), _row_spec(), _pair_spec(width=DEG_W),
        pl.BlockSpec((D, D), lambda i: (0, 0)),
        pl.BlockSpec((1, D), lambda i: (0, 0)),
    ],
    out_specs=_row_spec(),
    out_shape=jax.ShapeDtypeStruct((N_PAD, D), jnp.float32),
)


def kernel(x, edge_index, W, b):
    src = edge_index[0].astype(jnp.int32)
    dst = edge_index[1].astype(jnp.int32)
    pad = N_NODES + (jnp.arange(E_PAD - N_EDGES, dtype=jnp.int32)
                     % (N_PAD - N_NODES))
    srcp = jnp.concatenate([src, pad]).reshape(NW * CH, LANE)
    dstp = jnp.concatenate([dst, pad]).reshape(NW, CH, LANE)
    idxc = jnp.stack([srcp, dstp.reshape(NW * CH, LANE)], axis=1)
    x_p = jnp.pad(x, ((0, N_PAD - N_NODES), (0, 0)))
    degp = _deg_kernel(dstp)
    y = _scale(x_p, degp)
    zp = _hop_kernel(y, idxc)
    z = _mid(zp, y, degp)
    wp = _hop_kernel(z, idxc)
    out = _out(wp, z, degp, W, b.reshape(1, D))
    return out[:N_NODES]


# speedup vs baseline: 3.7399x; 1.0326x over previous
import functools

import jax
import jax.numpy as jnp
from jax import lax
from jax.experimental import pallas as pl
from jax.experimental.pallas import tpu as pltpu
from jax.experimental.pallas import tpu_sc as plsc

N_NODES = 10000
N_EDGES = 320000
D = 128

NC = 2
NS = 16
NW = NC * NS
LANE = 128
CH = 80
E_PAD = NW * CH * LANE
N_PAD = 10240
RPT = N_PAD // NS
DEG_W = 16

_mesh = plsc.VectorSubcoreMesh(
    core_axis_name="c", subcore_axis_name="s", num_cores=NC, num_subcores=NS)


@functools.partial(
    pl.kernel,
    out_type=jax.ShapeDtypeStruct((NC, N_PAD, DEG_W), jnp.float32),
    mesh=_mesh,
    scratch_types=[
        pltpu.VMEM((CH, LANE), jnp.int32),
        pltpu.VMEM((LANE, DEG_W), jnp.float32),
        pltpu.VMEM((LANE, DEG_W), jnp.float32),
        pltpu.VMEM_SHARED((N_PAD, DEG_W), jnp.float32),
    ],
)
def _deg_kernel(dstp_hbm, out_hbm, didx, ones, zeros, acc):
    c = lax.axis_index("c")
    s = lax.axis_index("s")
    wid = c * NS + s

    one16 = jnp.ones((16,), jnp.float32)
    zero16 = jnp.zeros((16,), jnp.float32)

    def fill(i, _):
        ones[i, pl.ds(0, 16)] = one16
        zeros[i, pl.ds(0, 16)] = zero16
        return 0
    lax.fori_loop(0, LANE, fill, 0)

    base = s * RPT
    for k in range(RPT // LANE):
        pltpu.sync_copy(zeros, acc.at[pl.ds(base + k * LANE, LANE)])
    plsc.subcore_barrier()

    pltpu.sync_copy(dstp_hbm.at[wid], didx)

    def body(j, _):
        pltpu.sync_copy(ones, acc.at[didx.at[j]], add=True)
        return 0
    lax.fori_loop(0, CH, body, 0)

    plsc.subcore_barrier()
    pltpu.sync_copy(acc.at[pl.ds(base, RPT)],
                    out_hbm.at[c, pl.ds(base, RPT)])


@functools.partial(
    pl.kernel,
    out_type=jax.ShapeDtypeStruct((NC, N_PAD, D), jnp.float32),
    mesh=_mesh,
    scratch_types=[
        pltpu.VMEM((2, LANE), jnp.int32),
        pltpu.VMEM((2, LANE), jnp.int32),
        pltpu.VMEM((LANE, D), jnp.float32),
        pltpu.VMEM((LANE, D), jnp.float32),
        pltpu.VMEM_SHARED((N_PAD, D), jnp.float32),
        pltpu.SemaphoreType.DMA,
        pltpu.SemaphoreType.DMA,
        pltpu.SemaphoreType.DMA,
        pltpu.SemaphoreType.DMA,
        pltpu.SemaphoreType.DMA,
        pltpu.SemaphoreType.DMA,
    ],
)
def _hop_kernel(h_hbm, idxc_hbm, out_hbm, slota, slotb, rowsa, rowsb,
                acc, sia, sib, sga, sgb, ssa, ssb):
    c = lax.axis_index("c")
    s = lax.axis_index("s")
    wid = c * NS + s
    cbase = wid * CH

    zero16 = jnp.zeros((16,), jnp.float32)

    with jax.named_scope("hop_zerofill"):
        def fill(i, _):
            def fill_in(j, _):
                rowsa[i, pl.ds(j * 16, 16)] = zero16
                return 0
            lax.fori_loop(0, D // 16, fill_in, 0)
            return 0
        lax.fori_loop(0, LANE, fill, 0)

    base = s * RPT
    with jax.named_scope("hop_zeroacc"):
        for k in range(RPT // LANE):
            pltpu.sync_copy(rowsa, acc.at[pl.ds(base + k * LANE, LANE)])
        plsc.subcore_barrier()

    with jax.named_scope("hop_edges"):
        def wait_idx(j, slot, sem):
            pltpu.make_async_copy(idxc_hbm.at[cbase + j], slot, sem).wait()

        def wait_gather(slot, buf, sem):
            pltpu.make_async_copy(h_hbm.at[slot.at[0]], buf, sem).wait()

        def wait_scatter(buf, slot, sem):
            pltpu.make_async_copy(buf, acc.at[slot.at[1]], sem).wait()

        pltpu.async_copy(idxc_hbm.at[cbase], slota, sia)
        wait_idx(0, slota, sia)
        pltpu.async_copy(h_hbm.at[slota.at[0]], rowsa, sga)

        def body(jj, _):
            j1 = 2 * jj + 1
            j2 = 2 * jj + 2

            @pl.when(jj > 0)
            def _():
                wait_scatter(rowsb, slotb, ssb)
            pltpu.async_copy(idxc_hbm.at[cbase + j1], slotb, sib)
            wait_gather(slota, rowsa, sga)
            pltpu.async_copy(rowsa, acc.at[slota.at[1]], ssa, add=True)
            wait_idx(j1, slotb, sib)
            pltpu.async_copy(h_hbm.at[slotb.at[0]], rowsb, sgb)
            wait_scatter(rowsa, slota, ssa)

            @pl.when(j2 < CH)
            def _():
                pltpu.async_copy(idxc_hbm.at[cbase + j2], slota, sia)
                wait_idx(j2, slota, sia)
                pltpu.async_copy(h_hbm.at[slota.at[0]], rowsa, sga)
            wait_gather(slotb, rowsb, sgb)
            pltpu.async_copy(rowsb, acc.at[slotb.at[1]], ssb, add=True)
            return 0
        lax.fori_loop(0, CH // 2, body, 0)
        wait_scatter(rowsb, slotb, ssb)
        plsc.subcore_barrier()

    with jax.named_scope("hop_writeout"):
        pltpu.sync_copy(acc.at[pl.ds(base, RPT)],
                        out_hbm.at[c, pl.ds(base, RPT)])


_BLK = 1024


def _scale_body(x_ref, d_ref, y_ref):
    deg = d_ref[0, :, 0:1] + d_ref[1, :, 0:1] + 1.0
    y_ref[...] = x_ref[...] * lax.rsqrt(deg)


def _mid_body(zp_ref, y_ref, d_ref, z_ref):
    deg = d_ref[0, :, 0:1] + d_ref[1, :, 0:1] + 1.0
    z_ref[...] = (zp_ref[0] + zp_ref[1] + y_ref[...]) / deg


def _out_body(wp_ref, z_ref, d_ref, w_ref, b_ref, o_ref):
    deg = d_ref[0, :, 0:1] + d_ref[1, :, 0:1] + 1.0
    w = (wp_ref[0] + wp_ref[1] + z_ref[...]) * lax.rsqrt(deg)
    o_ref[...] = (jnp.dot(w, w_ref[...], preferred_element_type=jnp.float32)
                  + b_ref[...])


def _row_spec(blk=_BLK, width=D):
    return pl.BlockSpec((blk, width), lambda i: (i, 0))


def _pair_spec(blk=_BLK, width=D):
    return pl.BlockSpec((2, blk, width), lambda i: (0, i, 0))


_GRID = (N_PAD // _BLK,)

_scale = pl.pallas_call(
    _scale_body,
    grid=_GRID,
    in_specs=[_row_spec(), _pair_spec(width=DEG_W)],
    out_specs=_row_spec(),
    out_shape=jax.ShapeDtypeStruct((N_PAD, D), jnp.float32),
)

_mid = pl.pallas_call(
    _mid_body,
    grid=_GRID,
    in_specs=[_pair_spec(), _row_spec(), _pair_spec(width=DEG_W)],
    out_specs=_row_spec(),
    out_shape=jax.ShapeDtypeStruct((N_PAD, D), jnp.float32),
)

_out = pl.pallas_call(
    _out_body,
    grid=_GRID,
    in_specs=[
        _pair_spec(), _row_spec(), _pair_spec(width=DEG_W),
        pl.BlockSpec((D, D), lambda i: (0, 0)),
        pl.BlockSpec((1, D), lambda i: (0, 0)),
    ],
    out_specs=_row_spec(),
    out_shape=jax.ShapeDtypeStruct((N_PAD, D), jnp.float32),
)


def kernel(x, edge_index, W, b):
    src = edge_index[0].astype(jnp.int32)
    dst = edge_index[1].astype(jnp.int32)
    pad = N_NODES + (jnp.arange(E_PAD - N_EDGES, dtype=jnp.int32)
                     % (N_PAD - N_NODES))
    srcp = jnp.concatenate([src, pad]).reshape(NW * CH, LANE)
    dstp = jnp.concatenate([dst, pad]).reshape(NW, CH, LANE)
    idxc = jnp.stack([srcp, dstp.reshape(NW * CH, LANE)], axis=1)
    x_p = jnp.pad(x, ((0, N_PAD - N_NODES), (0, 0)))
    degp = _deg_kernel(dstp)
    y = _scale(x_p, degp)
    zp = _hop_kernel(y, idxc)
    z = _mid(zp, y, degp)
    wp = _hop_kernel(z, idxc)
    out = _out(wp, z, degp, W, b.reshape(1, D))
    return out[:N_NODES]
